# baseline (speedup 1.0000x reference)
import functools

import jax
import jax.numpy as jnp
from jax import lax
from jax.experimental import pallas as pl
from jax.experimental.pallas import tpu as pltpu
from jax.experimental.pallas import tpu_sc as plsc

BT = 64
NB = 143
S = NB * BT
CH = 32
NBUF = 3


def _sc_scatter_x(x, slots_3d, S, IN):
    info = plsc.get_sparse_core_info()
    NC, NS = info.num_cores, info.num_subcores
    NW = NC * NS
    B = x.shape[0]
    b_per_w = B // NW
    n_chunks = b_per_w // CH
    mesh = plsc.VectorSubcoreMesh(core_axis_name="c", subcore_axis_name="s")

    @functools.partial(
        pl.kernel,
        mesh=mesh,
        out_type=jax.ShapeDtypeStruct((S, IN), jnp.float32),
        scratch_types=[
            pltpu.VMEM((n_chunks, CH), jnp.int32),
        ] + [pltpu.VMEM((CH, IN), jnp.float32) for _ in range(NBUF)]
          + [pltpu.SemaphoreType.DMA for _ in range(2 * NBUF)],
    )
    def scatter_k(x_hbm, idx_hbm, out_hbm, idx_v, *bufs_sems):
        rows_v = bufs_sems[:NBUF]
        semL = bufs_sems[NBUF:2 * NBUF]
        semS = bufs_sems[2 * NBUF:]
        wid = lax.axis_index("s") * NC + lax.axis_index("c")
        pltpu.sync_copy(idx_hbm.at[wid], idx_v)

        hL = [None] * NBUF
        hS = [None] * NBUF
        for k in range(n_chunks):
            j = k % NBUF
            if k >= NBUF:
                hS[j].wait()
            b0 = wid * b_per_w + k * CH
            hL[j] = pltpu.async_copy(x_hbm.at[pl.ds(b0, CH)], rows_v[j], semL[j])
            if k >= 1:
                jp = (k - 1) % NBUF
                hL[jp].wait()
                hS[jp] = pltpu.async_copy(
                    rows_v[jp], out_hbm.at[idx_v.at[k - 1]], semS[jp])
        jl = (n_chunks - 1) % NBUF
        hL[jl].wait()
        hS[jl] = pltpu.async_copy(
            rows_v[jl], out_hbm.at[idx_v.at[n_chunks - 1]], semS[jl])
        for k in range(max(0, n_chunks - NBUF), n_chunks):
            hS[k % NBUF].wait()

    return scatter_k(x, slots_3d)


def _sc_gather_y(y_sorted, slots_3d, B, OUT):
    info = plsc.get_sparse_core_info()
    NC, NS = info.num_cores, info.num_subcores
    NW = NC * NS
    b_per_w = B // NW
    n_chunks = b_per_w // CH
    mesh = plsc.VectorSubcoreMesh(core_axis_name="c", subcore_axis_name="s")

    @functools.partial(
        pl.kernel,
        mesh=mesh,
        out_type=jax.ShapeDtypeStruct((B, OUT), jnp.float32),
        scratch_types=[
            pltpu.VMEM((n_chunks, CH), jnp.int32),
        ] + [pltpu.VMEM((CH, OUT), jnp.float32) for _ in range(NBUF)]
          + [pltpu.SemaphoreType.DMA for _ in range(2 * NBUF)],
    )
    def gather_k(ys_hbm, idx_hbm, out_hbm, idx_v, *bufs_sems):
        rows_v = bufs_sems[:NBUF]
        semL = bufs_sems[NBUF:2 * NBUF]
        semS = bufs_sems[2 * NBUF:]
        wid = lax.axis_index("s") * NC + lax.axis_index("c")
        pltpu.sync_copy(idx_hbm.at[wid], idx_v)

        hL = [None] * NBUF
        hS = [None] * NBUF
        for k in range(n_chunks):
            j = k % NBUF
            if k >= NBUF:
                hS[j].wait()
            hL[j] = pltpu.async_copy(ys_hbm.at[idx_v.at[k]], rows_v[j], semL[j])
            if k >= 1:
                jp = (k - 1) % NBUF
                hL[jp].wait()
                b0 = wid * b_per_w + (k - 1) * CH
                hS[jp] = pltpu.async_copy(
                    rows_v[jp], out_hbm.at[pl.ds(b0, CH)], semS[jp])
        jl = (n_chunks - 1) % NBUF
        hL[jl].wait()
        hS[jl] = pltpu.async_copy(
            rows_v[jl],
            out_hbm.at[pl.ds(wid * b_per_w + (n_chunks - 1) * CH, CH)],
            semS[jl])
        for k in range(max(0, n_chunks - NBUF), n_chunks):
            hS[k % NBUF].wait()

    return gather_k(y_sorted, slots_3d)


def _mm_body(be_ref, x_ref, w_ref, b_ref, o_ref):
    acc = lax.dot_general(
        x_ref[...], w_ref[0],
        (((1,), (1,)), ((), ())),
        preferred_element_type=jnp.float32,
    )
    o_ref[...] = acc + b_ref[0]


def _tc_grouped_matmul(x_sorted, blk_e, weight, bias, OUT, IN):
    grid_spec = pltpu.PrefetchScalarGridSpec(
        num_scalar_prefetch=1,
        grid=(NB,),
        in_specs=[
            pl.BlockSpec((BT, IN), lambda i, be: (i, 0)),
            pl.BlockSpec((1, OUT, IN), lambda i, be: (be[i], 0, 0)),
            pl.BlockSpec((1, 1, OUT), lambda i, be: (be[i], 0, 0)),
        ],
        out_specs=pl.BlockSpec((BT, OUT), lambda i, be: (i, 0)),
    )
    return pl.pallas_call(
        _mm_body,
        grid_spec=grid_spec,
        out_shape=jax.ShapeDtypeStruct((S, OUT), jnp.float32),
        compiler_params=pltpu.CompilerParams(
            dimension_semantics=("arbitrary",),
        ),
    )(blk_e, x_sorted, weight, bias.reshape(bias.shape[0], 1, OUT))


def _routing(stack_idx, B, E):
    NCH = 64
    CL = B // NCH
    e2 = stack_idx.astype(jnp.int32).reshape(NCH, CL)
    ar_e = jnp.arange(E, dtype=jnp.int32)
    oh = (e2[:, None, :] == ar_e[None, :, None]).astype(jnp.float32)
    tri = (jnp.arange(CL)[:, None] <= jnp.arange(CL)[None, :]).astype(jnp.float32)
    within = lax.dot_general(
        oh.reshape(NCH * E, CL), tri, (((1,), (0,)), ((), ())),
        preferred_element_type=jnp.float32).reshape(NCH, E, CL)
    chunk_hist = within[:, :, -1].astype(jnp.int32)
    prefix = jnp.cumsum(chunk_hist, axis=0) - chunk_hist
    c = jnp.sum(chunk_hist, axis=0)

    nb = (c + BT - 1) // BT
    fb_end = jnp.cumsum(nb)
    fb = fb_end - nb
    blk = jnp.arange(NB, dtype=jnp.int32)
    blk_e = jnp.minimum(
        jnp.sum((blk[:, None] >= fb_end[None, :]).astype(jnp.int32), axis=1),
        E - 1)

    base = (fb[None, :] * BT + prefix).astype(jnp.float32)
    slots = jnp.sum((base[:, :, None] + within - 1.0) * oh, axis=1)
    return blk_e, slots.reshape(B).astype(jnp.int32)


def kernel(input, stack_idx, weight, bias):
    B, IN = input.shape
    E, OUT, _ = weight.shape
    blk_e, slots = _routing(stack_idx, B, E)

    info = plsc.get_sparse_core_info()
    NW = info.num_cores * info.num_subcores
    slots_3d = slots.reshape(NW, B // NW // CH, CH)

    x_sorted = _sc_scatter_x(input, slots_3d, S, IN)
    y_sorted = _tc_grouped_matmul(x_sorted, blk_e, weight, bias, OUT, IN)
    return _sc_gather_y(y_sorted, slots_3d, B, OUT)


# speedup vs baseline: 1.4993x; 1.4993x over previous
import functools

import jax
import jax.numpy as jnp
from jax import lax
from jax.experimental import pallas as pl
from jax.experimental.pallas import tpu as pltpu
from jax.experimental.pallas import tpu_sc as plsc

BT = 256
NB = 47
S = NB * BT
CH = 32
NBUF = 3


def _sc_scatter_x(x, slots_3d, S, IN):
    info = plsc.get_sparse_core_info()
    NC, NS = info.num_cores, info.num_subcores
    NW = NC * NS
    B = x.shape[0]
    b_per_w = B // NW
    n_chunks = b_per_w // CH
    mesh = plsc.VectorSubcoreMesh(core_axis_name="c", subcore_axis_name="s")

    @functools.partial(
        pl.kernel,
        mesh=mesh,
        out_type=jax.ShapeDtypeStruct((S, IN), jnp.float32),
        scratch_types=[
            pltpu.VMEM((n_chunks, CH), jnp.int32),
        ] + [pltpu.VMEM((CH, IN), jnp.float32) for _ in range(NBUF)]
          + [pltpu.SemaphoreType.DMA for _ in range(2 * NBUF)],
    )
    def scatter_k(x_hbm, idx_hbm, out_hbm, idx_v, *bufs_sems):
        rows_v = bufs_sems[:NBUF]
        semL = bufs_sems[NBUF:2 * NBUF]
        semS = bufs_sems[2 * NBUF:]
        wid = lax.axis_index("s") * NC + lax.axis_index("c")
        pltpu.sync_copy(idx_hbm.at[wid], idx_v)

        hL = [None] * NBUF
        hS = [None] * NBUF
        for k in range(n_chunks):
            j = k % NBUF
            if k >= NBUF:
                hS[j].wait()
            b0 = wid * b_per_w + k * CH
            hL[j] = pltpu.async_copy(x_hbm.at[pl.ds(b0, CH)], rows_v[j], semL[j])
            if k >= 1:
                jp = (k - 1) % NBUF
                hL[jp].wait()
                hS[jp] = pltpu.async_copy(
                    rows_v[jp], out_hbm.at[idx_v.at[k - 1]], semS[jp])
        jl = (n_chunks - 1) % NBUF
        hL[jl].wait()
        hS[jl] = pltpu.async_copy(
            rows_v[jl], out_hbm.at[idx_v.at[n_chunks - 1]], semS[jl])
        for k in range(max(0, n_chunks - NBUF), n_chunks):
            hS[k % NBUF].wait()

    return scatter_k(x, slots_3d)


def _sc_gather_y(y_sorted, slots_3d, B, OUT):
    info = plsc.get_sparse_core_info()
    NC, NS = info.num_cores, info.num_subcores
    NW = NC * NS
    b_per_w = B // NW
    n_chunks = b_per_w // CH
    mesh = plsc.VectorSubcoreMesh(core_axis_name="c", subcore_axis_name="s")

    @functools.partial(
        pl.kernel,
        mesh=mesh,
        out_type=jax.ShapeDtypeStruct((B, OUT), jnp.float32),
        scratch_types=[
            pltpu.VMEM((n_chunks, CH), jnp.int32),
        ] + [pltpu.VMEM((CH, OUT), jnp.float32) for _ in range(NBUF)]
          + [pltpu.SemaphoreType.DMA for _ in range(2 * NBUF)],
    )
    def gather_k(ys_hbm, idx_hbm, out_hbm, idx_v, *bufs_sems):
        rows_v = bufs_sems[:NBUF]
        semL = bufs_sems[NBUF:2 * NBUF]
        semS = bufs_sems[2 * NBUF:]
        wid = lax.axis_index("s") * NC + lax.axis_index("c")
        pltpu.sync_copy(idx_hbm.at[wid], idx_v)

        hL = [None] * NBUF
        hS = [None] * NBUF
        for k in range(n_chunks):
            j = k % NBUF
            if k >= NBUF:
                hS[j].wait()
            hL[j] = pltpu.async_copy(ys_hbm.at[idx_v.at[k]], rows_v[j], semL[j])
            if k >= 1:
                jp = (k - 1) % NBUF
                hL[jp].wait()
                b0 = wid * b_per_w + (k - 1) * CH
                hS[jp] = pltpu.async_copy(
                    rows_v[jp], out_hbm.at[pl.ds(b0, CH)], semS[jp])
        jl = (n_chunks - 1) % NBUF
        hL[jl].wait()
        hS[jl] = pltpu.async_copy(
            rows_v[jl],
            out_hbm.at[pl.ds(wid * b_per_w + (n_chunks - 1) * CH, CH)],
            semS[jl])
        for k in range(max(0, n_chunks - NBUF), n_chunks):
            hS[k % NBUF].wait()

    return gather_k(y_sorted, slots_3d)


def _mm_body(be_ref, x_ref, w_ref, b_ref, o_ref):
    acc = lax.dot_general(
        x_ref[...], w_ref[0],
        (((1,), (1,)), ((), ())),
        preferred_element_type=jnp.float32,
    )
    o_ref[...] = acc + b_ref[0]


def _tc_grouped_matmul(x_sorted, blk_e, weight, bias, OUT, IN):
    grid_spec = pltpu.PrefetchScalarGridSpec(
        num_scalar_prefetch=1,
        grid=(NB,),
        in_specs=[
            pl.BlockSpec((BT, IN), lambda i, be: (i, 0)),
            pl.BlockSpec((1, OUT, IN), lambda i, be: (be[i], 0, 0)),
            pl.BlockSpec((1, 1, OUT), lambda i, be: (be[i], 0, 0)),
        ],
        out_specs=pl.BlockSpec((BT, OUT), lambda i, be: (i, 0)),
    )
    return pl.pallas_call(
        _mm_body,
        grid_spec=grid_spec,
        out_shape=jax.ShapeDtypeStruct((S, OUT), jnp.float32),
        compiler_params=pltpu.CompilerParams(
            dimension_semantics=("arbitrary",),
        ),
    )(blk_e, x_sorted, weight, bias.reshape(bias.shape[0], 1, OUT))


def _routing(stack_idx, B, E):
    NCH = 64
    CL = B // NCH
    e2 = stack_idx.astype(jnp.int32).reshape(NCH, CL)
    ar_e = jnp.arange(E, dtype=jnp.int32)
    oh = (e2[:, None, :] == ar_e[None, :, None]).astype(jnp.float32)
    tri = (jnp.arange(CL)[:, None] <= jnp.arange(CL)[None, :]).astype(jnp.float32)
    within = lax.dot_general(
        oh.reshape(NCH * E, CL), tri, (((1,), (0,)), ((), ())),
        preferred_element_type=jnp.float32).reshape(NCH, E, CL)
    chunk_hist = within[:, :, -1].astype(jnp.int32)
    prefix = jnp.cumsum(chunk_hist, axis=0) - chunk_hist
    c = jnp.sum(chunk_hist, axis=0)

    nb = (c + BT - 1) // BT
    fb_end = jnp.cumsum(nb)
    fb = fb_end - nb
    blk = jnp.arange(NB, dtype=jnp.int32)
    blk_e = jnp.minimum(
        jnp.sum((blk[:, None] >= fb_end[None, :]).astype(jnp.int32), axis=1),
        E - 1)

    base = (fb[None, :] * BT + prefix).astype(jnp.float32)
    slots = jnp.sum((base[:, :, None] + within - 1.0) * oh, axis=1)
    return blk_e, slots.reshape(B).astype(jnp.int32)


def kernel(input, stack_idx, weight, bias):
    B, IN = input.shape
    E, OUT, _ = weight.shape
    blk_e, slots = _routing(stack_idx, B, E)

    info = plsc.get_sparse_core_info()
    NW = info.num_cores * info.num_subcores
    slots_3d = slots.reshape(NW, B // NW // CH, CH)

    x_sorted = _sc_scatter_x(input, slots_3d, S, IN)
    y_sorted = _tc_grouped_matmul(x_sorted, blk_e, weight, bias, OUT, IN)
    return _sc_gather_y(y_sorted, slots_3d, B, OUT)
